# baseline (speedup 1.0000x reference)
import functools

import jax
import jax.numpy as jnp
from jax import lax
from jax.experimental import pallas as pl
from jax.experimental.pallas import tpu as pltpu
from jax.experimental.pallas import tpu_sc as plsc

N = 100000
C = 128
B = 16384
NC = 2
NS = 16
NW = NC * NS
RANGE = 6256
NPAD = RANGE * NS
CHUNK = B // NW
SUB = 128
NSUB = CHUNK // SUB


@functools.partial(
    pl.kernel,
    out_type=jax.ShapeDtypeStruct((B, C), jnp.float32),
    mesh=plsc.VectorSubcoreMesh(core_axis_name="c", subcore_axis_name="s"),
    compiler_params=pltpu.CompilerParams(needs_layout_passes=False),
    scratch_types=[
        pltpu.VMEM((B,), jnp.int32),
        pltpu.VMEM((CHUNK,), jnp.int32),
        pltpu.VMEM((RANGE,), jnp.int32),
        pltpu.VMEM_SHARED((B,), jnp.int32),
        pltpu.VMEM_SHARED((NPAD,), jnp.int32),
        pltpu.VMEM((CHUNK,), jnp.int32),
        pltpu.VMEM((2, SUB, C), jnp.float32),
        pltpu.VMEM((2, SUB, C), jnp.float32),
        pltpu.VMEM((2, SUB, C), jnp.float32),
        pltpu.SemaphoreType.DMA,
        pltpu.SemaphoreType.DMA,
        pltpu.SemaphoreType.DMA,
    ],
)
def _sc_kernel(probs_hbm, idx_hbm, latent_hbm, out_hbm,
               idx_v, idxc_v, win_v, stage_s, table_s, w_v, l_v, p_v, o_v,
               sem, sem_l, sem_o):
    sid = lax.axis_index("s")
    wid = sid * NC + lax.axis_index("c")
    base = wid * CHUNK
    lo = sid * RANGE
    slice_b = B // NS

    with jax.named_scope("idxcopy"):
        pltpu.sync_copy(idx_hbm.at[pl.ds(base, CHUNK)], idxc_v)
    cp_l = [None] * NSUB
    cp_l[0] = pltpu.async_copy(
        latent_hbm.at[idxc_v.at[pl.ds(0, SUB)]], l_v.at[0], sem_l)
    with jax.named_scope("idxstage"):
        pltpu.sync_copy(idx_hbm.at[pl.ds(sid * slice_b, slice_b)],
                        stage_s.at[pl.ds(sid * slice_b, slice_b)])
        plsc.subcore_barrier()
        pltpu.sync_copy(stage_s, idx_v)

    def body(j, carry):
        for u in range(8):
            off = j * 128 + u * 16
            v = idx_v[pl.ds(off, 16)]
            rel_u = plsc.bitcast(v - lo, jnp.uint32)
            mask = rel_u < jnp.uint32(RANGE)
            rel = plsc.bitcast(
                jnp.minimum(rel_u, jnp.uint32(RANGE - 1)), jnp.int32)
            b = lax.iota(jnp.int32, 16) + off
            plsc.store_scatter(win_v, [rel], b, mask=mask)
        return carry

    with jax.named_scope("phaseA"):
        lax.fori_loop(0, B // 128, body, 0)
    with jax.named_scope("publish"):
        pltpu.sync_copy(win_v, table_s.at[pl.ds(lo, RANGE)])
        plsc.subcore_barrier()

    with jax.named_scope("wgather"):
        pltpu.async_copy(table_s.at[idxc_v], w_v, sem).wait()

    cp_p = [None] * NSUB
    cp_o = [None] * NSUB
    HALF = SUB // 2
    cp_p0a = pltpu.async_copy(
        probs_hbm.at[w_v.at[pl.ds(0, HALF)]], p_v.at[0].at[pl.ds(0, HALF)],
        sem)
    cp_p[0] = pltpu.async_copy(
        probs_hbm.at[w_v.at[pl.ds(HALF, HALF)]],
        p_v.at[0].at[pl.ds(HALF, HALF)], sem)

    for s in range(NSUB):
        if s + 1 < NSUB:
            cp_l[s + 1] = pltpu.async_copy(
                latent_hbm.at[idxc_v.at[pl.ds((s + 1) * SUB, SUB)]],
                l_v.at[(s + 1) % 2], sem_l)
            cp_p[s + 1] = pltpu.async_copy(
                probs_hbm.at[w_v.at[pl.ds((s + 1) * SUB, SUB)]],
                p_v.at[(s + 1) % 2], sem)
        with jax.named_scope(f"dmawait{s}"):
            if s == 0:
                cp_p0a.wait()
            else:
                cp_p[s].wait()
            cp_l[s].wait()
            if s >= 2:
                cp_o[s - 2].wait()
        pb = p_v.at[s % 2]
        lb = l_v.at[s % 2]
        ob = o_v.at[s % 2]

        def rowpair(i, carry):
            one = jnp.full((16,), 1.0, jnp.float32)
            rows = [2 * i, 2 * i + 1]
            segs = [[], []]
            rs1 = [None, None]
            for k in range(2):
                tot = jnp.zeros((16,), jnp.float32)
                for j in range(C // 16):
                    v = pb[rows[k], pl.ds(j * 16, 16)]
                    v = jnp.clip(v, 0.0001, 1.0 - 0.0001)
                    segs[k].append(v)
                    tot = tot + v
                rs1[k] = (1.0 - 0.9) * (
                    one / jnp.broadcast_to(jnp.sum(tot), (16,)))
            sq = [[], []]
            rs2 = [None, None]
            for k in range(2):
                tot2 = jnp.zeros((16,), jnp.float32)
                for j in range(C // 16):
                    g = lb[rows[k], pl.ds(j * 16, 16)]
                    nr = 0.9 * g + segs[k][j] * rs1[k]
                    q = nr * nr
                    sq[k].append(q)
                    tot2 = tot2 + q
                rs2[k] = one / jnp.broadcast_to(jnp.sum(tot2), (16,))
            for k in range(2):
                for j in range(C // 16):
                    ob[rows[k], pl.ds(j * 16, 16)] = sq[k][j] * rs2[k]
            return carry

        with jax.named_scope(f"compute{s}"):
            if s == 0:
                lax.fori_loop(0, HALF // 2, rowpair, 0)
                cp_p[0].wait()
                lax.fori_loop(HALF // 2, SUB // 2, rowpair, 0)
            else:
                lax.fori_loop(0, SUB // 2, rowpair, 0)
        cp_o[s] = pltpu.async_copy(
            ob, out_hbm.at[pl.ds(base + s * SUB, SUB)], sem_o)

    cp_o[NSUB - 2].wait()
    cp_o[NSUB - 1].wait()


def kernel(probs, index, latent):
    return _sc_kernel(probs, index, latent)


# speedup vs baseline: 1.1656x; 1.1656x over previous
import functools

import jax
import jax.numpy as jnp
from jax import lax
from jax.experimental import pallas as pl
from jax.experimental.pallas import tpu as pltpu
from jax.experimental.pallas import tpu_sc as plsc

N = 100000
C = 128
B = 16384
NC = 2
NS = 16
NW = NC * NS
RANGE = 6256
NPAD = RANGE * NS
CHUNK = B // NW
SUB = 128
NSUB = CHUNK // SUB


@functools.partial(
    pl.kernel,
    out_type=jax.ShapeDtypeStruct((B, C), jnp.float32),
    mesh=plsc.VectorSubcoreMesh(core_axis_name="c", subcore_axis_name="s"),
    compiler_params=pltpu.CompilerParams(needs_layout_passes=False),
    scratch_types=[
        pltpu.VMEM((B,), jnp.int32),
        pltpu.VMEM((CHUNK,), jnp.int32),
        pltpu.VMEM((RANGE,), jnp.int32),
        pltpu.VMEM_SHARED((B,), jnp.int32),
        pltpu.VMEM_SHARED((NPAD,), jnp.int32),
        pltpu.VMEM((CHUNK,), jnp.int32),
        pltpu.VMEM((2, SUB, C), jnp.float32),
        pltpu.VMEM((2, SUB, C), jnp.float32),
        pltpu.VMEM((2, SUB, C), jnp.float32),
        pltpu.SemaphoreType.DMA,
        pltpu.SemaphoreType.DMA,
        pltpu.SemaphoreType.DMA,
    ],
)
def _sc_kernel(probs_hbm, idx_hbm, latent_hbm, out_hbm,
               idx_v, idxc_v, win_v, stage_s, table_s, w_v, l_v, p_v, o_v,
               sem, sem_l, sem_o):
    sid = lax.axis_index("s")
    wid = sid * NC + lax.axis_index("c")
    base = wid * CHUNK
    lo = sid * RANGE
    slice_b = B // NS

    with jax.named_scope("idxcopy"):
        pltpu.sync_copy(idx_hbm.at[pl.ds(base, CHUNK)], idxc_v)
    cp_l = [None] * NSUB
    cp_l[0] = pltpu.async_copy(
        latent_hbm.at[idxc_v.at[pl.ds(0, SUB)]], l_v.at[0], sem_l)
    with jax.named_scope("idxstage"):
        pltpu.sync_copy(idx_hbm.at[pl.ds(sid * slice_b, slice_b)],
                        stage_s.at[pl.ds(sid * slice_b, slice_b)])
        plsc.subcore_barrier()
        pltpu.sync_copy(stage_s, idx_v)

    def body(j, carry):
        vs = [idx_v[pl.ds(j * 128 + u * 16, 16)] for u in range(8)]
        rels, masks, bs = [], [], []
        for u in range(8):
            off = j * 128 + u * 16
            rel_u = plsc.bitcast(vs[u] - lo, jnp.uint32)
            masks.append(rel_u < jnp.uint32(RANGE))
            rels.append(plsc.bitcast(
                jnp.minimum(rel_u, jnp.uint32(RANGE - 1)), jnp.int32))
            bs.append(lax.iota(jnp.int32, 16) + off)
        for u in range(8):
            plsc.store_scatter(win_v, [rels[u]], bs[u], mask=masks[u])
        return carry

    with jax.named_scope("phaseA"):
        lax.fori_loop(0, B // 128, body, 0)
    with jax.named_scope("publish"):
        pltpu.sync_copy(win_v, table_s.at[pl.ds(lo, RANGE)])
        plsc.subcore_barrier()

    with jax.named_scope("wgather"):
        pltpu.async_copy(table_s.at[idxc_v], w_v, sem).wait()

    cp_p = [None] * NSUB
    cp_o = [None] * NSUB
    HALF = SUB // 2
    cp_p0a = pltpu.async_copy(
        probs_hbm.at[w_v.at[pl.ds(0, HALF)]], p_v.at[0].at[pl.ds(0, HALF)],
        sem)
    cp_p[0] = pltpu.async_copy(
        probs_hbm.at[w_v.at[pl.ds(HALF, HALF)]],
        p_v.at[0].at[pl.ds(HALF, HALF)], sem)

    for s in range(NSUB):
        if s + 1 < NSUB:
            cp_l[s + 1] = pltpu.async_copy(
                latent_hbm.at[idxc_v.at[pl.ds((s + 1) * SUB, SUB)]],
                l_v.at[(s + 1) % 2], sem_l)
            cp_p[s + 1] = pltpu.async_copy(
                probs_hbm.at[w_v.at[pl.ds((s + 1) * SUB, SUB)]],
                p_v.at[(s + 1) % 2], sem)
        with jax.named_scope(f"dmawait{s}"):
            if s == 0:
                cp_p0a.wait()
            else:
                cp_p[s].wait()
            cp_l[s].wait()
            if s >= 2:
                cp_o[s - 2].wait()
        pb = p_v.at[s % 2]
        lb = l_v.at[s % 2]
        ob = o_v.at[s % 2]

        def rowpair(i, carry):
            one = jnp.full((16,), 1.0, jnp.float32)
            rows = [2 * i, 2 * i + 1]
            segs = [[], []]
            rs1 = [None, None]
            for k in range(2):
                tot = jnp.zeros((16,), jnp.float32)
                for j in range(C // 16):
                    v = pb[rows[k], pl.ds(j * 16, 16)]
                    v = jnp.clip(v, 0.0001, 1.0 - 0.0001)
                    segs[k].append(v)
                    tot = tot + v
                rs1[k] = (1.0 - 0.9) * (
                    one / jnp.broadcast_to(jnp.sum(tot), (16,)))
            sq = [[], []]
            rs2 = [None, None]
            for k in range(2):
                tot2 = jnp.zeros((16,), jnp.float32)
                for j in range(C // 16):
                    g = lb[rows[k], pl.ds(j * 16, 16)]
                    nr = 0.9 * g + segs[k][j] * rs1[k]
                    q = nr * nr
                    sq[k].append(q)
                    tot2 = tot2 + q
                rs2[k] = one / jnp.broadcast_to(jnp.sum(tot2), (16,))
            for k in range(2):
                for j in range(C // 16):
                    ob[rows[k], pl.ds(j * 16, 16)] = sq[k][j] * rs2[k]
            return carry

        with jax.named_scope(f"compute{s}"):
            if s == 0:
                lax.fori_loop(0, HALF // 2, rowpair, 0)
                cp_p[0].wait()
                lax.fori_loop(HALF // 2, SUB // 2, rowpair, 0)
            else:
                lax.fori_loop(0, SUB // 2, rowpair, 0)
        cp_o[s] = pltpu.async_copy(
            ob, out_hbm.at[pl.ds(base + s * SUB, SUB)], sem_o)

    cp_o[NSUB - 2].wait()
    cp_o[NSUB - 1].wait()


def kernel(probs, index, latent):
    return _sc_kernel(probs, index, latent)


# speedup vs baseline: 1.1772x; 1.0100x over previous
import functools

import jax
import jax.numpy as jnp
from jax import lax
from jax.experimental import pallas as pl
from jax.experimental.pallas import tpu as pltpu
from jax.experimental.pallas import tpu_sc as plsc

N = 100000
C = 128
B = 16384
NC = 2
NS = 16
NW = NC * NS
RANGE = 6256
NPAD = RANGE * NS
CHUNK = B // NW
SUB = 128
NSUB = CHUNK // SUB


@functools.partial(
    pl.kernel,
    out_type=jax.ShapeDtypeStruct((B, C), jnp.float32),
    mesh=plsc.VectorSubcoreMesh(core_axis_name="c", subcore_axis_name="s"),
    compiler_params=pltpu.CompilerParams(needs_layout_passes=False),
    scratch_types=[
        pltpu.VMEM((B,), jnp.int32),
        pltpu.VMEM((CHUNK,), jnp.int32),
        pltpu.VMEM((RANGE,), jnp.int32),
        pltpu.VMEM_SHARED((B,), jnp.int32),
        pltpu.VMEM_SHARED((NPAD,), jnp.int32),
        pltpu.VMEM((CHUNK,), jnp.int32),
        pltpu.VMEM((2, SUB, C), jnp.float32),
        pltpu.VMEM((2, SUB, C), jnp.float32),
        pltpu.VMEM((2, SUB, C), jnp.float32),
        pltpu.SemaphoreType.DMA,
        pltpu.SemaphoreType.DMA,
        pltpu.SemaphoreType.DMA,
    ],
)
def _sc_kernel(probs_hbm, idx_hbm, latent_hbm, out_hbm,
               idx_v, idxc_v, win_v, stage_s, table_s, w_v, l_v, p_v, o_v,
               sem, sem_l, sem_o):
    sid = lax.axis_index("s")
    wid = sid * NC + lax.axis_index("c")
    base = wid * CHUNK
    lo = sid * RANGE
    slice_b = B // NS

    with jax.named_scope("idxcopy"):
        pltpu.sync_copy(idx_hbm.at[pl.ds(base, CHUNK)], idxc_v)
    cp_l = [None] * NSUB
    cp_l[0] = pltpu.async_copy(
        latent_hbm.at[idxc_v.at[pl.ds(0, SUB)]], l_v.at[0], sem_l)
    with jax.named_scope("idxstage"):
        pltpu.sync_copy(idx_hbm.at[pl.ds(sid * slice_b, slice_b)],
                        stage_s.at[pl.ds(sid * slice_b, slice_b)])
        plsc.subcore_barrier()
        pltpu.sync_copy(stage_s, idx_v)

    def body(j, carry):
        vs = [idx_v[pl.ds(j * 128 + u * 16, 16)] for u in range(8)]
        rels, masks, bs = [], [], []
        for u in range(8):
            off = j * 128 + u * 16
            rel_u = plsc.bitcast(vs[u] - lo, jnp.uint32)
            masks.append(rel_u < jnp.uint32(RANGE))
            rels.append(plsc.bitcast(
                jnp.minimum(rel_u, jnp.uint32(RANGE - 1)), jnp.int32))
            bs.append(lax.iota(jnp.int32, 16) + off)
        for u in range(8):
            plsc.store_scatter(win_v, [rels[u]], bs[u], mask=masks[u])
        return carry

    with jax.named_scope("phaseA"):
        lax.fori_loop(0, B // 128, body, 0)
    with jax.named_scope("publish"):
        pltpu.sync_copy(win_v, table_s.at[pl.ds(lo, RANGE)])
        plsc.subcore_barrier()

    with jax.named_scope("wgather"):
        pltpu.async_copy(table_s.at[idxc_v], w_v, sem).wait()

    cp_p = [None] * NSUB
    cp_o = [None] * NSUB
    HALF = SUB // 2
    cp_p0a = pltpu.async_copy(
        probs_hbm.at[w_v.at[pl.ds(0, HALF)]], p_v.at[0].at[pl.ds(0, HALF)],
        sem)
    cp_p[0] = pltpu.async_copy(
        probs_hbm.at[w_v.at[pl.ds(HALF, HALF)]],
        p_v.at[0].at[pl.ds(HALF, HALF)], sem)

    for s in range(NSUB):
        if s + 1 < NSUB:
            cp_l[s + 1] = pltpu.async_copy(
                latent_hbm.at[idxc_v.at[pl.ds((s + 1) * SUB, SUB)]],
                l_v.at[(s + 1) % 2], sem_l)
            cp_p[s + 1] = pltpu.async_copy(
                probs_hbm.at[w_v.at[pl.ds((s + 1) * SUB, SUB)]],
                p_v.at[(s + 1) % 2], sem)
        with jax.named_scope(f"dmawait{s}"):
            if s == 0:
                cp_p0a.wait()
            else:
                cp_p[s].wait()
            cp_l[s].wait()
            if s >= 2:
                cp_o[s - 2].wait()
        pb = p_v.at[s % 2]
        lb = l_v.at[s % 2]
        ob = o_v.at[s % 2]

        def rowpair(i, carry):
            one = jnp.full((16,), 1.0, jnp.float32)
            rows = [2 * i, 2 * i + 1]
            ps = [[pb[rows[k], pl.ds(j * 16, 16)] for j in range(C // 16)]
                  for k in range(2)]
            gs = [[lb[rows[k], pl.ds(j * 16, 16)] for j in range(C // 16)]
                  for k in range(2)]
            segs = [[], []]
            rs1 = [None, None]
            for k in range(2):
                tot = jnp.zeros((16,), jnp.float32)
                for j in range(C // 16):
                    v = jnp.clip(ps[k][j], 0.0001, 1.0 - 0.0001)
                    segs[k].append(v)
                    tot = tot + v
                rs1[k] = (1.0 - 0.9) * (
                    one / jnp.broadcast_to(jnp.sum(tot), (16,)))
            sq = [[], []]
            rs2 = [None, None]
            for k in range(2):
                tot2 = jnp.zeros((16,), jnp.float32)
                for j in range(C // 16):
                    nr = 0.9 * gs[k][j] + segs[k][j] * rs1[k]
                    q = nr * nr
                    sq[k].append(q)
                    tot2 = tot2 + q
                rs2[k] = one / jnp.broadcast_to(jnp.sum(tot2), (16,))
            for k in range(2):
                for j in range(C // 16):
                    ob[rows[k], pl.ds(j * 16, 16)] = sq[k][j] * rs2[k]
            return carry

        with jax.named_scope(f"compute{s}"):
            if s == 0:
                lax.fori_loop(0, HALF // 2, rowpair, 0)
                cp_p[0].wait()
                lax.fori_loop(HALF // 2, SUB // 2, rowpair, 0)
            else:
                lax.fori_loop(0, SUB // 2, rowpair, 0)
        cp_o[s] = pltpu.async_copy(
            ob, out_hbm.at[pl.ds(base + s * SUB, SUB)], sem_o)

    cp_o[NSUB - 2].wait()
    cp_o[NSUB - 1].wait()


def kernel(probs, index, latent):
    return _sc_kernel(probs, index, latent)
